# baseline (speedup 1.0000x reference)
import functools

import jax
import jax.numpy as jnp
from jax import lax
from jax.experimental import pallas as pl
from jax.experimental.pallas import tpu as pltpu
from jax.experimental.pallas import tpu_sc as plsc

N = 10000
E = 160000
D = 256
H = 8
HD = 32

NP = 10240
EP = 163840
NBLK = EP // 128
STRIPE = NP // 16

f32 = jnp.float32
i32 = jnp.int32

_MESH = plsc.VectorSubcoreMesh(core_axis_name="c", subcore_axis_name="s")
_SC_PARAMS = pltpu.CompilerParams(
    needs_layout_passes=False, use_tc_tiling_on_sc=False
)


def _node_body(x_ref, wt_ref, a12_ref, h4_ref, src_ref):
    t = pl.program_id(0)
    x = x_ref[...]
    h = jnp.dot(x, wt_ref[...], preferred_element_type=f32)
    rows = t * 1024 + lax.broadcasted_iota(i32, (1024, 1), 0)
    h = jnp.where(rows < N, h, 0.0)
    for q in range(4):
        h4_ref[q, :, :] = h[:, 64 * q:64 * q + 64]
    src_ref[...] = jnp.dot(h, a12_ref[...], preferred_element_type=f32)


def _k_node(feats, wt, a12):
    return pl.pallas_call(
        _node_body,
        grid=(10,),
        in_specs=[
            pl.BlockSpec((1024, 256), lambda t: (t, 0)),
            pl.BlockSpec((256, 256), lambda t: (0, 0)),
            pl.BlockSpec((256, 16), lambda t: (0, 0)),
        ],
        out_specs=[
            pl.BlockSpec((4, 1024, 64), lambda t: (0, t, 0)),
            pl.BlockSpec((1024, 16), lambda t: (t, 0)),
        ],
        out_shape=[
            jax.ShapeDtypeStruct((4, NP, 64), f32),
            jax.ShapeDtypeStruct((NP, 16), f32),
        ],
    )(feats, wt, a12)


def _edge_body(x_ref, wt_ref, b8_ref, ea_ref, t_ref):
    ea = jnp.dot(x_ref[...], wt_ref[...], preferred_element_type=f32)
    ea_ref[...] = ea
    t_ref[...] = jnp.dot(ea, b8_ref[...], preferred_element_type=f32)


def _k_edge(edge_attr, wt, b8):
    return pl.pallas_call(
        _edge_body,
        grid=(80,),
        in_specs=[
            pl.BlockSpec((2000, 16), lambda t: (t, 0)),
            pl.BlockSpec((16, 16), lambda t: (0, 0)),
            pl.BlockSpec((16, 8), lambda t: (0, 0)),
        ],
        out_specs=[
            pl.BlockSpec((2000, 16), lambda t: (t, 0)),
            pl.BlockSpec((2000, 8), lambda t: (t, 0)),
        ],
        out_shape=[
            jax.ShapeDtypeStruct((E, 16), f32),
            jax.ShapeDtypeStruct((E, 8), f32),
        ],
    )(edge_attr, wt, b8)


def _rden_body(dp_ref, r_ref, rexp_ref):
    r = 1.0 / (dp_ref[0, :, :] + dp_ref[1, :, :])
    rexp_ref[...] = jnp.dot(r, r_ref[...], preferred_element_type=f32)


def _k_rden(dp3, rmat):
    return pl.pallas_call(
        _rden_body,
        grid=(10,),
        in_specs=[
            pl.BlockSpec((2, 1024, 8), lambda t: (0, t, 0)),
            pl.BlockSpec((8, 256), lambda t: (0, 0)),
        ],
        out_specs=pl.BlockSpec((1024, 256), lambda t: (t, 0)),
        out_shape=jax.ShapeDtypeStruct((NP, 256), f32),
    )(dp3, rmat)


def _out_body(a0_ref, a1_ref, a2_ref, a3_ref, rexp_ref, wt_ref, b_ref, bo_ref,
              o_ref):
    x = jnp.concatenate(
        [a0_ref[...], a1_ref[...], a2_ref[...], a3_ref[...]], axis=1)
    x = x * rexp_ref[...] + b_ref[...]
    o_ref[...] = jnp.dot(x, wt_ref[...], preferred_element_type=f32) + bo_ref[...]


def _k_out(a4, rexp, wt, b, bo):
    return pl.pallas_call(
        _out_body,
        grid=(10,),
        in_specs=[
            pl.BlockSpec((1024, 64), lambda t: (t, 0)),
            pl.BlockSpec((1024, 64), lambda t: (t, 0)),
            pl.BlockSpec((1024, 64), lambda t: (t, 0)),
            pl.BlockSpec((1024, 64), lambda t: (t, 0)),
            pl.BlockSpec((1024, 256), lambda t: (t, 0)),
            pl.BlockSpec((256, 256), lambda t: (0, 0)),
            pl.BlockSpec((1, 256), lambda t: (0, 0)),
            pl.BlockSpec((1, 256), lambda t: (0, 0)),
        ],
        out_specs=pl.BlockSpec((1024, 256), lambda t: (t, 0)),
        out_shape=jax.ShapeDtypeStruct((N, 256), f32),
    )(a4[0], a4[1], a4[2], a4[3], rexp, wt, b, bo)


NB_A = NBLK // 32
NB_G = NBLK // 16


def _attn_body(row_hbm, col_hbm, t_hbm, src_hbm, z8_hbm, ex_hbm, dp_hbm,
               rowslab, colflat, tslab, srgs, scgs, exflats, ex2ds, acc,
               gsems, ssems):
    c = lax.axis_index("c")
    s = lax.axis_index("s")
    w = c * 16 + s
    pltpu.sync_copy(z8_hbm.at[pl.ds(s * STRIPE, STRIPE), :],
                    acc.at[pl.ds(s * STRIPE, STRIPE), :])
    pltpu.sync_copy(row_hbm.at[pl.ds(w * NB_A, NB_A), :], rowslab)
    pltpu.sync_copy(col_hbm.at[pl.ds(w * NB_A, NB_A), :], colflat)
    pltpu.sync_copy(t_hbm.at[pl.ds(w * NB_A * 1024, NB_A * 1024)], tslab)
    plsc.subcore_barrier()

    iota = lax.iota(i32, 16)
    id8 = iota // 8
    im8 = iota % 8

    @pl.loop(0, NB_A)
    def _blk(k):
        p = 0
        g = w * NB_A + k
        pltpu.sync_copy(src_hbm.at[rowslab.at[k]], srgs[p])
        pltpu.sync_copy(src_hbm.at[colflat.at[k]], scgs[p])

        @pl.loop(0, 64, unroll=4)
        def _vec(v):
            ir = 2 * v + id8
            srv = plsc.load_gather(srgs[p], [ir, im8])
            scv = plsc.load_gather(scgs[p], [ir, im8 + 8])
            tv = tslab[pl.ds(1024 * k + 16 * v, 16)]
            lg = srv + scv + tv
            exv = jnp.exp(jnp.maximum(lg, 0.01 * lg))
            exflats[p][pl.ds(16 * v, 16)] = exv
            plsc.store_scatter(ex2ds[p], [ir, im8], exv)

        pltpu.sync_copy(ex2ds[p], acc.at[rowslab.at[k]], add=True)
        pltpu.sync_copy(exflats[p], ex_hbm.at[pl.ds(g * 1024, 1024)])

    plsc.subcore_barrier()
    pltpu.sync_copy(acc.at[pl.ds(s * STRIPE, STRIPE), :],
                    dp_hbm.at[c].at[pl.ds(s * STRIPE, STRIPE), :])


def _k_attn(row2d, col2d, tp, src_tab, z8):
    return pl.kernel(
        _attn_body,
        out_type=(
            jax.ShapeDtypeStruct((EP * 8,), f32),
            jax.ShapeDtypeStruct((2, NP, 8), f32),
        ),
        mesh=_MESH,
        compiler_params=_SC_PARAMS,
        scratch_types=[
            pltpu.VMEM((NB_A, 128), i32),
            pltpu.VMEM((NB_A, 128), i32),
            pltpu.VMEM((NB_A * 1024,), f32),
            [pltpu.VMEM((128, 16), f32)] * 2,
            [pltpu.VMEM((128, 16), f32)] * 2,
            [pltpu.VMEM((1024,), f32)] * 2,
            [pltpu.VMEM((128, 8), f32)] * 2,
            pltpu.VMEM_SHARED((NP, 8), f32),
            [pltpu.SemaphoreType.DMA] * 2,
            [pltpu.SemaphoreType.DMA] * 2,
        ],
    )(row2d, col2d, tp, src_tab, z8)


def _agg_body(row_hbm, col_hbm, ex_hbm, h4_hbm, z64_hbm, agg_hbm,
              rowslab, colflat2, hbufs, msgbufs, exbs, acc, gsems, ssems):
    c = lax.axis_index("c")
    s = lax.axis_index("s")
    pltpu.sync_copy(row_hbm.at[pl.ds(s * NB_G, NB_G), :], rowslab)
    pltpu.sync_copy(col_hbm.at[pl.ds(s * NB_G * 128, NB_G * 128)], colflat2)

    iota = lax.iota(i32, 16)
    izero = iota * 0

    off0 = 2 * c * NP

    @pl.loop(0, NB_G * 8, unroll=4)
    def _shift(v):
        colflat2[pl.ds(16 * v, 16)] = colflat2[pl.ds(16 * v, 16)] + off0

    for q in range(2):
        if q == 1:
            @pl.loop(0, NB_G * 8, unroll=4)
            def _shift2(v):
                colflat2[pl.ds(16 * v, 16)] = colflat2[pl.ds(16 * v, 16)] + NP

        pltpu.sync_copy(z64_hbm.at[pl.ds(s * STRIPE, STRIPE), :],
                        acc.at[pl.ds(s * STRIPE, STRIPE), :])
        plsc.subcore_barrier()

        def g_descs(k, p):
            return [
                pltpu.make_async_copy(
                    h4_hbm.at[colflat2.at[pl.ds(128 * k, 128)]],
                    hbufs[p], gsems[p]),
                pltpu.make_async_copy(
                    ex_hbm.at[pl.ds((s * NB_G + k) * 1024, 1024)],
                    exbs[p], gsems[p]),
            ]

        def s_desc(k, p):
            return pltpu.make_async_copy(msgbufs[p], acc.at[rowslab.at[k]],
                                         ssems[p])

        h0 = 4 * c + 2 * q

        @pl.loop(0, NB_G)
        def _blk(k):
            p = 0
            pltpu.sync_copy(h4_hbm.at[colflat2.at[pl.ds(128 * k, 128)]],
                            hbufs[p])
            pltpu.sync_copy(ex_hbm.at[pl.ds((s * NB_G + k) * 1024, 1024)],
                            exbs[p])

            @pl.loop(0, 128, unroll=4)
            def _edge(i):
                base = 8 * i + h0
                for jj in range(2):
                    av = plsc.load_gather(exbs[p], [(base + jj) + izero])
                    lo = 32 * jj
                    msgbufs[p][i, pl.ds(lo, 16)] = (
                        hbufs[p][i, pl.ds(lo, 16)] * av)
                    msgbufs[p][i, pl.ds(lo + 16, 16)] = (
                        hbufs[p][i, pl.ds(lo + 16, 16)] * av)

            pltpu.sync_copy(msgbufs[p], acc.at[rowslab.at[k]], add=True)

        plsc.subcore_barrier()
        pltpu.sync_copy(acc.at[pl.ds(s * STRIPE, STRIPE), :],
                        agg_hbm.at[2 * c + q].at[pl.ds(s * STRIPE, STRIPE), :])


def _k_agg(row2d, col_flat, ex_flat, h4flat, z64):
    return pl.kernel(
        _agg_body,
        out_type=jax.ShapeDtypeStruct((4, NP, 64), f32),
        mesh=_MESH,
        compiler_params=_SC_PARAMS,
        scratch_types=[
            pltpu.VMEM((NB_G, 128), i32),
            pltpu.VMEM((NB_G * 128,), i32),
            [pltpu.VMEM((128, 64), f32)] * 2,
            [pltpu.VMEM((128, 64), f32)] * 2,
            [pltpu.VMEM((1024,), f32)] * 2,
            pltpu.VMEM_SHARED((NP, 64), f32),
            [pltpu.SemaphoreType.DMA] * 2,
            [pltpu.SemaphoreType.DMA] * 2,
        ],
    )(row2d, col_flat, ex_flat, h4flat, z64)


@jax.jit
def kernel(feats, edge_index, edge_attr, W_fc, W_edge, att, bias, W_out, b_out):
    row = edge_index[:, 0]
    col = edge_index[:, 1]
    rowp = jnp.concatenate([row, jnp.full((EP - E,), N, i32)]).reshape(NBLK, 128)
    colp = jnp.concatenate([col, jnp.full((EP - E,), N, i32)]).reshape(NBLK, 128)

    att_f = att[..., 0]
    r256 = jnp.arange(256)
    a12 = (
        jnp.zeros((256, 16), f32)
        .at[r256, r256 // 32].set(att_f[:, :32].reshape(-1))
        .at[r256, 8 + r256 // 32].set(att_f[:, 32:64].reshape(-1))
    )
    r16 = jnp.arange(16)
    b8 = jnp.zeros((16, 8), f32).at[r16, r16 // 2].set(att_f[:, 64:66].reshape(-1))

    r64 = jnp.arange(256)
    rmat = jnp.zeros((8, 256), f32).at[r64 // 32, r64].set(1.0)

    h4, src_tab = _k_node(feats, W_fc.T, a12)
    ea16, t8 = _k_edge(edge_attr, W_edge.T, b8)
    tp = jnp.pad(t8.reshape(-1), (0, (EP - E) * 8))

    z8 = jnp.zeros((NP, 8), f32)
    z64 = jnp.zeros((NP, 64), f32)

    ex_flat, dpart = _k_attn(rowp, colp, tp, src_tab, z8)
    rexp = _k_rden(dpart, rmat)
    agg4 = _k_agg(rowp, colp.reshape(-1), ex_flat,
                  h4.reshape(4 * NP, 64), z64)

    out = _k_out(agg4, rexp, W_out.T, bias.reshape(1, 256),
                 b_out.reshape(1, 256))
    return out, edge_index, ea16


# speedup vs baseline: 1.3496x; 1.3496x over previous
import functools

import jax
import jax.numpy as jnp
from jax import lax
from jax.experimental import pallas as pl
from jax.experimental.pallas import tpu as pltpu
from jax.experimental.pallas import tpu_sc as plsc

N = 10000
E = 160000
D = 256
H = 8
HD = 32

NP = 10240
EP = 163840
NBLK = EP // 128
STRIPE = NP // 16

f32 = jnp.float32
i32 = jnp.int32

_MESH = plsc.VectorSubcoreMesh(core_axis_name="c", subcore_axis_name="s")
_SC_PARAMS = pltpu.CompilerParams(
    needs_layout_passes=False, use_tc_tiling_on_sc=False
)


def _node_body(x_ref, wt_ref, a12_ref, h4_ref, src_ref):
    t = pl.program_id(0)
    x = x_ref[...]
    h = jnp.dot(x, wt_ref[...], preferred_element_type=f32)
    rows = t * 1024 + lax.broadcasted_iota(i32, (1024, 1), 0)
    h = jnp.where(rows < N, h, 0.0)
    for q in range(4):
        h4_ref[q, :, :] = h[:, 64 * q:64 * q + 64]
    src_ref[...] = jnp.dot(h, a12_ref[...], preferred_element_type=f32)


def _k_node(feats, wt, a12):
    return pl.pallas_call(
        _node_body,
        grid=(10,),
        in_specs=[
            pl.BlockSpec((1024, 256), lambda t: (t, 0)),
            pl.BlockSpec((256, 256), lambda t: (0, 0)),
            pl.BlockSpec((256, 16), lambda t: (0, 0)),
        ],
        out_specs=[
            pl.BlockSpec((4, 1024, 64), lambda t: (0, t, 0)),
            pl.BlockSpec((1024, 16), lambda t: (t, 0)),
        ],
        out_shape=[
            jax.ShapeDtypeStruct((4, NP, 64), f32),
            jax.ShapeDtypeStruct((NP, 16), f32),
        ],
    )(feats, wt, a12)


def _edge_body(x_ref, wt_ref, b8_ref, ea_ref, t_ref):
    ea = jnp.dot(x_ref[...], wt_ref[...], preferred_element_type=f32)
    ea_ref[...] = ea
    t_ref[...] = jnp.dot(ea, b8_ref[...], preferred_element_type=f32)


def _k_edge(edge_attr, wt, b8):
    return pl.pallas_call(
        _edge_body,
        grid=(80,),
        in_specs=[
            pl.BlockSpec((2000, 16), lambda t: (t, 0)),
            pl.BlockSpec((16, 16), lambda t: (0, 0)),
            pl.BlockSpec((16, 8), lambda t: (0, 0)),
        ],
        out_specs=[
            pl.BlockSpec((2000, 16), lambda t: (t, 0)),
            pl.BlockSpec((2000, 8), lambda t: (t, 0)),
        ],
        out_shape=[
            jax.ShapeDtypeStruct((E, 16), f32),
            jax.ShapeDtypeStruct((E, 8), f32),
        ],
    )(edge_attr, wt, b8)


def _rden_body(dp_ref, r_ref, rexp_ref):
    r = 1.0 / (dp_ref[0, :, :] + dp_ref[1, :, :])
    rexp_ref[...] = jnp.dot(r, r_ref[...], preferred_element_type=f32)


def _k_rden(dp3, rmat):
    return pl.pallas_call(
        _rden_body,
        grid=(10,),
        in_specs=[
            pl.BlockSpec((2, 1024, 8), lambda t: (0, t, 0)),
            pl.BlockSpec((8, 256), lambda t: (0, 0)),
        ],
        out_specs=pl.BlockSpec((1024, 256), lambda t: (t, 0)),
        out_shape=jax.ShapeDtypeStruct((NP, 256), f32),
    )(dp3, rmat)


def _out_body(a0_ref, a1_ref, a2_ref, a3_ref, rexp_ref, wt_ref, b_ref, bo_ref,
              o_ref):
    x = jnp.concatenate(
        [a0_ref[...], a1_ref[...], a2_ref[...], a3_ref[...]], axis=1)
    x = x * rexp_ref[...] + b_ref[...]
    o_ref[...] = jnp.dot(x, wt_ref[...], preferred_element_type=f32) + bo_ref[...]


def _k_out(a4, rexp, wt, b, bo):
    return pl.pallas_call(
        _out_body,
        grid=(10,),
        in_specs=[
            pl.BlockSpec((1024, 64), lambda t: (t, 0)),
            pl.BlockSpec((1024, 64), lambda t: (t, 0)),
            pl.BlockSpec((1024, 64), lambda t: (t, 0)),
            pl.BlockSpec((1024, 64), lambda t: (t, 0)),
            pl.BlockSpec((1024, 256), lambda t: (t, 0)),
            pl.BlockSpec((256, 256), lambda t: (0, 0)),
            pl.BlockSpec((1, 256), lambda t: (0, 0)),
            pl.BlockSpec((1, 256), lambda t: (0, 0)),
        ],
        out_specs=pl.BlockSpec((1024, 256), lambda t: (t, 0)),
        out_shape=jax.ShapeDtypeStruct((N, 256), f32),
    )(a4[0], a4[1], a4[2], a4[3], rexp, wt, b, bo)


NB_A = NBLK // 32
NB_G = NBLK // 16


def _attn_body(row_hbm, col_hbm, t_hbm, src_hbm, z8_hbm, ex_hbm, dp_hbm,
               rowslab, colflat, tslab, srgs, scgs, exflats, ex2ds, acc,
               gsems, ssems):
    c = lax.axis_index("c")
    s = lax.axis_index("s")
    w = c * 16 + s
    pltpu.sync_copy(z8_hbm.at[pl.ds(s * STRIPE, STRIPE), :],
                    acc.at[pl.ds(s * STRIPE, STRIPE), :])
    pltpu.sync_copy(row_hbm.at[pl.ds(w * NB_A, NB_A), :], rowslab)
    pltpu.sync_copy(col_hbm.at[pl.ds(w * NB_A, NB_A), :], colflat)
    pltpu.sync_copy(t_hbm.at[pl.ds(w * NB_A * 1024, NB_A * 1024)], tslab)
    plsc.subcore_barrier()

    iota = lax.iota(i32, 16)
    id8 = iota // 8
    im8 = iota % 8

    @pl.loop(0, NB_A)
    def _blk(k):
        p = 0
        g = w * NB_A + k
        pltpu.sync_copy(src_hbm.at[rowslab.at[k]], srgs[p])
        pltpu.sync_copy(src_hbm.at[colflat.at[k]], scgs[p])

        @pl.loop(0, 64, unroll=4)
        def _vec(v):
            ir = 2 * v + id8
            srv = plsc.load_gather(srgs[p], [ir, im8])
            scv = plsc.load_gather(scgs[p], [ir, im8 + 8])
            tv = tslab[pl.ds(1024 * k + 16 * v, 16)]
            lg = srv + scv + tv
            exv = jnp.exp(jnp.maximum(lg, 0.01 * lg))
            exflats[p][pl.ds(16 * v, 16)] = exv
            plsc.store_scatter(ex2ds[p], [ir, im8], exv)

        pltpu.sync_copy(ex2ds[p], acc.at[rowslab.at[k]], add=True)
        pltpu.sync_copy(exflats[p], ex_hbm.at[pl.ds(g * 1024, 1024)])

    plsc.subcore_barrier()
    pltpu.sync_copy(acc.at[pl.ds(s * STRIPE, STRIPE), :],
                    dp_hbm.at[c].at[pl.ds(s * STRIPE, STRIPE), :])


def _k_attn(row2d, col2d, tp, src_tab, z8):
    return pl.kernel(
        _attn_body,
        out_type=(
            jax.ShapeDtypeStruct((EP * 8,), f32),
            jax.ShapeDtypeStruct((2, NP, 8), f32),
        ),
        mesh=_MESH,
        compiler_params=_SC_PARAMS,
        scratch_types=[
            pltpu.VMEM((NB_A, 128), i32),
            pltpu.VMEM((NB_A, 128), i32),
            pltpu.VMEM((NB_A * 1024,), f32),
            [pltpu.VMEM((128, 16), f32)] * 2,
            [pltpu.VMEM((128, 16), f32)] * 2,
            [pltpu.VMEM((1024,), f32)] * 2,
            [pltpu.VMEM((128, 8), f32)] * 2,
            pltpu.VMEM_SHARED((NP, 8), f32),
            [pltpu.SemaphoreType.DMA] * 2,
            [pltpu.SemaphoreType.DMA] * 2,
        ],
    )(row2d, col2d, tp, src_tab, z8)


def _agg_body(row_hbm, col_hbm, ex_hbm, h4_hbm, z64_hbm, agg_hbm,
              rowslab, colflat2, hbufs, msgbufs, exbs, acc, gsems, ssems):
    c = lax.axis_index("c")
    s = lax.axis_index("s")
    pltpu.sync_copy(row_hbm.at[pl.ds(s * NB_G, NB_G), :], rowslab)
    pltpu.sync_copy(col_hbm.at[pl.ds(s * NB_G * 128, NB_G * 128)], colflat2)

    iota = lax.iota(i32, 16)
    izero = iota * 0

    off0 = 2 * c * NP

    @pl.loop(0, NB_G * 8, unroll=4)
    def _shift(v):
        colflat2[pl.ds(16 * v, 16)] = colflat2[pl.ds(16 * v, 16)] + off0

    for q in range(2):
        if q == 1:
            @pl.loop(0, NB_G * 8, unroll=4)
            def _shift2(v):
                colflat2[pl.ds(16 * v, 16)] = colflat2[pl.ds(16 * v, 16)] + NP

        pltpu.sync_copy(z64_hbm.at[pl.ds(s * STRIPE, STRIPE), :],
                        acc.at[pl.ds(s * STRIPE, STRIPE), :])
        plsc.subcore_barrier()

        def g_descs(k, p):
            return [
                pltpu.make_async_copy(
                    h4_hbm.at[colflat2.at[pl.ds(128 * k, 128)]],
                    hbufs[p], gsems[p]),
                pltpu.make_async_copy(
                    ex_hbm.at[pl.ds((s * NB_G + k) * 1024, 1024)],
                    exbs[p], gsems[p]),
            ]

        def s_desc(k, p):
            return pltpu.make_async_copy(msgbufs[p], acc.at[rowslab.at[k]],
                                         ssems[p])

        h0 = 4 * c + 2 * q

        def step(k, p):
            for d in g_descs(k, p):
                d.wait()

            @pl.loop(0, 128, unroll=4)
            def _edge(i):
                base = 8 * i + h0
                for jj in range(2):
                    av = plsc.load_gather(exbs[p], [(base + jj) + izero])
                    lo = 32 * jj
                    msgbufs[0][i, pl.ds(lo, 16)] = (
                        hbufs[p][i, pl.ds(lo, 16)] * av)
                    msgbufs[0][i, pl.ds(lo + 16, 16)] = (
                        hbufs[p][i, pl.ds(lo + 16, 16)] * av)

            pltpu.sync_copy(msgbufs[0], acc.at[rowslab.at[k]], add=True)
            knext = jnp.minimum(k + 2, NB_G - 1)
            for d in g_descs(knext, p):
                d.start()

        for p in range(2):
            for d in g_descs(p, p):
                d.start()

        @pl.loop(0, NB_G // 2)
        def _pair(kk):
            step(2 * kk, 0)
            step(2 * kk + 1, 1)

        for p in range(2):
            for d in g_descs(NB_G - 1, p):
                d.wait()

        plsc.subcore_barrier()
        pltpu.sync_copy(acc.at[pl.ds(s * STRIPE, STRIPE), :],
                        agg_hbm.at[2 * c + q].at[pl.ds(s * STRIPE, STRIPE), :])


def _k_agg(row2d, col_flat, ex_flat, h4flat, z64):
    return pl.kernel(
        _agg_body,
        out_type=jax.ShapeDtypeStruct((4, NP, 64), f32),
        mesh=_MESH,
        compiler_params=_SC_PARAMS,
        scratch_types=[
            pltpu.VMEM((NB_G, 128), i32),
            pltpu.VMEM((NB_G * 128,), i32),
            [pltpu.VMEM((128, 64), f32)] * 2,
            [pltpu.VMEM((128, 64), f32)] * 2,
            [pltpu.VMEM((1024,), f32)] * 2,
            pltpu.VMEM_SHARED((NP, 64), f32),
            [pltpu.SemaphoreType.DMA] * 2,
            [pltpu.SemaphoreType.DMA] * 2,
        ],
    )(row2d, col_flat, ex_flat, h4flat, z64)


@jax.jit
def kernel(feats, edge_index, edge_attr, W_fc, W_edge, att, bias, W_out, b_out):
    row = edge_index[:, 0]
    col = edge_index[:, 1]
    rowp = jnp.concatenate([row, jnp.full((EP - E,), N, i32)]).reshape(NBLK, 128)
    colp = jnp.concatenate([col, jnp.full((EP - E,), N, i32)]).reshape(NBLK, 128)

    att_f = att[..., 0]
    r256 = jnp.arange(256)
    a12 = (
        jnp.zeros((256, 16), f32)
        .at[r256, r256 // 32].set(att_f[:, :32].reshape(-1))
        .at[r256, 8 + r256 // 32].set(att_f[:, 32:64].reshape(-1))
    )
    r16 = jnp.arange(16)
    b8 = jnp.zeros((16, 8), f32).at[r16, r16 // 2].set(att_f[:, 64:66].reshape(-1))

    r64 = jnp.arange(256)
    rmat = jnp.zeros((8, 256), f32).at[r64 // 32, r64].set(1.0)

    h4, src_tab = _k_node(feats, W_fc.T, a12)
    ea16, t8 = _k_edge(edge_attr, W_edge.T, b8)
    tp = jnp.pad(t8.reshape(-1), (0, (EP - E) * 8))

    z8 = jnp.zeros((NP, 8), f32)
    z64 = jnp.zeros((NP, 64), f32)

    ex_flat, dpart = _k_attn(rowp, colp, tp, src_tab, z8)
    rexp = _k_rden(dpart, rmat)
    agg4 = _k_agg(rowp, colp.reshape(-1), ex_flat,
                  h4.reshape(4 * NP, 64), z64)

    out = _k_out(agg4, rexp, W_out.T, bias.reshape(1, 256),
                 b_out.reshape(1, 256))
    return out, edge_index, ea16


# speedup vs baseline: 1.4503x; 1.0746x over previous
import functools

import jax
import jax.numpy as jnp
from jax import lax
from jax.experimental import pallas as pl
from jax.experimental.pallas import tpu as pltpu
from jax.experimental.pallas import tpu_sc as plsc

N = 10000
E = 160000
D = 256
H = 8
HD = 32

NP = 10240
EP = 163840
NBLK = EP // 128
STRIPE = NP // 16

f32 = jnp.float32
i32 = jnp.int32

_MESH = plsc.VectorSubcoreMesh(core_axis_name="c", subcore_axis_name="s")
_SC_PARAMS = pltpu.CompilerParams(
    needs_layout_passes=False, use_tc_tiling_on_sc=False
)


def _node_body(x_ref, wt_ref, a12_ref, h4_ref, src_ref):
    t = pl.program_id(0)
    x = x_ref[...]
    h = jnp.dot(x, wt_ref[...], preferred_element_type=f32)
    rows = t * 1024 + lax.broadcasted_iota(i32, (1024, 1), 0)
    h = jnp.where(rows < N, h, 0.0)
    for q in range(4):
        h4_ref[q, :, :] = h[:, 64 * q:64 * q + 64]
    src_ref[...] = jnp.dot(h, a12_ref[...], preferred_element_type=f32)


def _k_node(feats, wt, a12):
    return pl.pallas_call(
        _node_body,
        grid=(10,),
        in_specs=[
            pl.BlockSpec((1024, 256), lambda t: (t, 0)),
            pl.BlockSpec((256, 256), lambda t: (0, 0)),
            pl.BlockSpec((256, 16), lambda t: (0, 0)),
        ],
        out_specs=[
            pl.BlockSpec((4, 1024, 64), lambda t: (0, t, 0)),
            pl.BlockSpec((1024, 16), lambda t: (t, 0)),
        ],
        out_shape=[
            jax.ShapeDtypeStruct((4, NP, 64), f32),
            jax.ShapeDtypeStruct((NP, 16), f32),
        ],
    )(feats, wt, a12)


def _edge_body(x_ref, wt_ref, b8_ref, ea_ref, t_ref):
    ea = jnp.dot(x_ref[...], wt_ref[...], preferred_element_type=f32)
    ea_ref[...] = ea
    t_ref[...] = jnp.dot(ea, b8_ref[...], preferred_element_type=f32)


def _k_edge(edge_attr, wt, b8):
    return pl.pallas_call(
        _edge_body,
        grid=(80,),
        in_specs=[
            pl.BlockSpec((2000, 16), lambda t: (t, 0)),
            pl.BlockSpec((16, 16), lambda t: (0, 0)),
            pl.BlockSpec((16, 8), lambda t: (0, 0)),
        ],
        out_specs=[
            pl.BlockSpec((2000, 16), lambda t: (t, 0)),
            pl.BlockSpec((2000, 8), lambda t: (t, 0)),
        ],
        out_shape=[
            jax.ShapeDtypeStruct((E, 16), f32),
            jax.ShapeDtypeStruct((E, 8), f32),
        ],
    )(edge_attr, wt, b8)


def _rden_body(dp_ref, r_ref, rexp_ref):
    r = 1.0 / (dp_ref[0, :, :] + dp_ref[1, :, :])
    rexp_ref[...] = jnp.dot(r, r_ref[...], preferred_element_type=f32)


def _k_rden(dp3, rmat):
    return pl.pallas_call(
        _rden_body,
        grid=(10,),
        in_specs=[
            pl.BlockSpec((2, 1024, 8), lambda t: (0, t, 0)),
            pl.BlockSpec((8, 256), lambda t: (0, 0)),
        ],
        out_specs=pl.BlockSpec((1024, 256), lambda t: (t, 0)),
        out_shape=jax.ShapeDtypeStruct((NP, 256), f32),
    )(dp3, rmat)


def _out_body(a0_ref, a1_ref, a2_ref, a3_ref, rexp_ref, wt_ref, b_ref, bo_ref,
              o_ref):
    x = jnp.concatenate(
        [a0_ref[...], a1_ref[...], a2_ref[...], a3_ref[...]], axis=1)
    x = x * rexp_ref[...] + b_ref[...]
    o_ref[...] = jnp.dot(x, wt_ref[...], preferred_element_type=f32) + bo_ref[...]


def _k_out(a4, rexp, wt, b, bo):
    return pl.pallas_call(
        _out_body,
        grid=(10,),
        in_specs=[
            pl.BlockSpec((1024, 64), lambda t: (t, 0)),
            pl.BlockSpec((1024, 64), lambda t: (t, 0)),
            pl.BlockSpec((1024, 64), lambda t: (t, 0)),
            pl.BlockSpec((1024, 64), lambda t: (t, 0)),
            pl.BlockSpec((1024, 256), lambda t: (t, 0)),
            pl.BlockSpec((256, 256), lambda t: (0, 0)),
            pl.BlockSpec((1, 256), lambda t: (0, 0)),
            pl.BlockSpec((1, 256), lambda t: (0, 0)),
        ],
        out_specs=pl.BlockSpec((1024, 256), lambda t: (t, 0)),
        out_shape=jax.ShapeDtypeStruct((N, 256), f32),
    )(a4[0], a4[1], a4[2], a4[3], rexp, wt, b, bo)


NB_A = NBLK // 32
NB_G = NBLK // 16


def _attn_body(row_hbm, col_hbm, t_hbm, src_hbm, z8_hbm, ex_hbm, dp_hbm,
               rowslab, colflat, tslab, srgs, scgs, exflats, ex2ds, acc,
               gsems, ssems):
    c = lax.axis_index("c")
    s = lax.axis_index("s")
    w = c * 16 + s
    pltpu.sync_copy(z8_hbm.at[pl.ds(s * STRIPE, STRIPE), :],
                    acc.at[pl.ds(s * STRIPE, STRIPE), :])
    pltpu.sync_copy(row_hbm.at[pl.ds(w * NB_A, NB_A), :], rowslab)
    pltpu.sync_copy(col_hbm.at[pl.ds(w * NB_A, NB_A), :], colflat)
    pltpu.sync_copy(t_hbm.at[pl.ds(w * NB_A * 1024, NB_A * 1024)], tslab)
    plsc.subcore_barrier()

    iota = lax.iota(i32, 16)
    id8 = iota // 8
    im8 = iota % 8

    def g_descs(k, p):
        return [
            pltpu.make_async_copy(src_hbm.at[rowslab.at[k]], srgs[p], gsems[p]),
            pltpu.make_async_copy(src_hbm.at[colflat.at[k]], scgs[p], gsems[p]),
        ]

    def step(k, p):
        for d in g_descs(k, p):
            d.wait()

        @pl.loop(0, 64, unroll=4)
        def _vec(v):
            ir = 2 * v + id8
            srv = plsc.load_gather(srgs[p], [ir, im8])
            scv = plsc.load_gather(scgs[p], [ir, im8 + 8])
            tv = tslab[pl.ds(1024 * k + 16 * v, 16)]
            lg = srv + scv + tv
            exv = jnp.exp(jnp.maximum(lg, 0.01 * lg))
            exflats[0][pl.ds(16 * v, 16)] = exv
            plsc.store_scatter(ex2ds[0], [ir, im8], exv)

        pltpu.sync_copy(ex2ds[0], acc.at[rowslab.at[k]], add=True)
        pltpu.sync_copy(exflats[0],
                        ex_hbm.at[pl.ds((w * NB_A + k) * 1024, 1024)])
        knext = jnp.minimum(k + 2, NB_A - 1)
        for d in g_descs(knext, p):
            d.start()

    for p in range(2):
        for d in g_descs(p, p):
            d.start()

    @pl.loop(0, NB_A // 2)
    def _pair(kk):
        step(2 * kk, 0)
        step(2 * kk + 1, 1)

    for p in range(2):
        for d in g_descs(NB_A - 1, p):
            d.wait()

    plsc.subcore_barrier()
    pltpu.sync_copy(acc.at[pl.ds(s * STRIPE, STRIPE), :],
                    dp_hbm.at[c].at[pl.ds(s * STRIPE, STRIPE), :])


def _k_attn(row2d, col2d, tp, src_tab, z8):
    return pl.kernel(
        _attn_body,
        out_type=(
            jax.ShapeDtypeStruct((EP * 8,), f32),
            jax.ShapeDtypeStruct((2, NP, 8), f32),
        ),
        mesh=_MESH,
        compiler_params=_SC_PARAMS,
        scratch_types=[
            pltpu.VMEM((NB_A, 128), i32),
            pltpu.VMEM((NB_A, 128), i32),
            pltpu.VMEM((NB_A * 1024,), f32),
            [pltpu.VMEM((128, 16), f32)] * 2,
            [pltpu.VMEM((128, 16), f32)] * 2,
            [pltpu.VMEM((1024,), f32)] * 2,
            [pltpu.VMEM((128, 8), f32)] * 2,
            pltpu.VMEM_SHARED((NP, 8), f32),
            [pltpu.SemaphoreType.DMA] * 2,
            [pltpu.SemaphoreType.DMA] * 2,
        ],
    )(row2d, col2d, tp, src_tab, z8)


def _agg_body(row_hbm, col_hbm, ex_hbm, h4_hbm, z64_hbm, agg_hbm,
              rowslab, colflat2, hbufs, msgbufs, exbs, acc, gsems, ssems):
    c = lax.axis_index("c")
    s = lax.axis_index("s")
    pltpu.sync_copy(row_hbm.at[pl.ds(s * NB_G, NB_G), :], rowslab)
    pltpu.sync_copy(col_hbm.at[pl.ds(s * NB_G * 128, NB_G * 128)], colflat2)

    iota = lax.iota(i32, 16)
    izero = iota * 0

    off0 = 2 * c * NP

    @pl.loop(0, NB_G * 8, unroll=4)
    def _shift(v):
        colflat2[pl.ds(16 * v, 16)] = colflat2[pl.ds(16 * v, 16)] + off0

    for q in range(2):
        if q == 1:
            @pl.loop(0, NB_G * 8, unroll=4)
            def _shift2(v):
                colflat2[pl.ds(16 * v, 16)] = colflat2[pl.ds(16 * v, 16)] + NP

        pltpu.sync_copy(z64_hbm.at[pl.ds(s * STRIPE, STRIPE), :],
                        acc.at[pl.ds(s * STRIPE, STRIPE), :])
        plsc.subcore_barrier()

        def g_descs(k, p):
            return [
                pltpu.make_async_copy(
                    h4_hbm.at[colflat2.at[pl.ds(128 * k, 128)]],
                    hbufs[p], gsems[p]),
                pltpu.make_async_copy(
                    ex_hbm.at[pl.ds((s * NB_G + k) * 1024, 1024)],
                    exbs[p], gsems[p]),
            ]

        def s_desc(k, p):
            return pltpu.make_async_copy(msgbufs[p], acc.at[rowslab.at[k]],
                                         ssems[p])

        h0 = 4 * c + 2 * q

        def step(k, p):
            for d in g_descs(k, p):
                d.wait()

            @pl.loop(0, 128, unroll=4)
            def _edge(i):
                base = 8 * i + h0
                for jj in range(2):
                    av = plsc.load_gather(exbs[p], [(base + jj) + izero])
                    lo = 32 * jj
                    msgbufs[0][i, pl.ds(lo, 16)] = (
                        hbufs[p][i, pl.ds(lo, 16)] * av)
                    msgbufs[0][i, pl.ds(lo + 16, 16)] = (
                        hbufs[p][i, pl.ds(lo + 16, 16)] * av)

            pltpu.sync_copy(msgbufs[0], acc.at[rowslab.at[k]], add=True)
            knext = jnp.minimum(k + 2, NB_G - 1)
            for d in g_descs(knext, p):
                d.start()

        for p in range(2):
            for d in g_descs(p, p):
                d.start()

        @pl.loop(0, NB_G // 2)
        def _pair(kk):
            step(2 * kk, 0)
            step(2 * kk + 1, 1)

        for p in range(2):
            for d in g_descs(NB_G - 1, p):
                d.wait()

        plsc.subcore_barrier()
        pltpu.sync_copy(acc.at[pl.ds(s * STRIPE, STRIPE), :],
                        agg_hbm.at[2 * c + q].at[pl.ds(s * STRIPE, STRIPE), :])


def _k_agg(row2d, col_flat, ex_flat, h4flat, z64):
    return pl.kernel(
        _agg_body,
        out_type=jax.ShapeDtypeStruct((4, NP, 64), f32),
        mesh=_MESH,
        compiler_params=_SC_PARAMS,
        scratch_types=[
            pltpu.VMEM((NB_G, 128), i32),
            pltpu.VMEM((NB_G * 128,), i32),
            [pltpu.VMEM((128, 64), f32)] * 2,
            [pltpu.VMEM((128, 64), f32)] * 2,
            [pltpu.VMEM((1024,), f32)] * 2,
            pltpu.VMEM_SHARED((NP, 64), f32),
            [pltpu.SemaphoreType.DMA] * 2,
            [pltpu.SemaphoreType.DMA] * 2,
        ],
    )(row2d, col_flat, ex_flat, h4flat, z64)


@jax.jit
def kernel(feats, edge_index, edge_attr, W_fc, W_edge, att, bias, W_out, b_out):
    row = edge_index[:, 0]
    col = edge_index[:, 1]
    rowp = jnp.concatenate([row, jnp.full((EP - E,), N, i32)]).reshape(NBLK, 128)
    colp = jnp.concatenate([col, jnp.full((EP - E,), N, i32)]).reshape(NBLK, 128)

    att_f = att[..., 0]
    r256 = jnp.arange(256)
    a12 = (
        jnp.zeros((256, 16), f32)
        .at[r256, r256 // 32].set(att_f[:, :32].reshape(-1))
        .at[r256, 8 + r256 // 32].set(att_f[:, 32:64].reshape(-1))
    )
    r16 = jnp.arange(16)
    b8 = jnp.zeros((16, 8), f32).at[r16, r16 // 2].set(att_f[:, 64:66].reshape(-1))

    r64 = jnp.arange(256)
    rmat = jnp.zeros((8, 256), f32).at[r64 // 32, r64].set(1.0)

    h4, src_tab = _k_node(feats, W_fc.T, a12)
    ea16, t8 = _k_edge(edge_attr, W_edge.T, b8)
    tp = jnp.pad(t8.reshape(-1), (0, (EP - E) * 8))

    z8 = jnp.zeros((NP, 8), f32)
    z64 = jnp.zeros((NP, 64), f32)

    ex_flat, dpart = _k_attn(rowp, colp, tp, src_tab, z8)
    rexp = _k_rden(dpart, rmat)
    agg4 = _k_agg(rowp, colp.reshape(-1), ex_flat,
                  h4.reshape(4 * NP, 64), z64)

    out = _k_out(agg4, rexp, W_out.T, bias.reshape(1, 256),
                 b_out.reshape(1, 256))
    return out, edge_index, ea16


# speedup vs baseline: 1.6136x; 1.1126x over previous
import functools

import jax
import jax.numpy as jnp
from jax import lax
from jax.experimental import pallas as pl
from jax.experimental.pallas import tpu as pltpu
from jax.experimental.pallas import tpu_sc as plsc

N = 10000
E = 160000
D = 256
H = 8
HD = 32

NP = 10240
EP = 163840
NBLK = EP // 128
STRIPE = NP // 16

f32 = jnp.float32
i32 = jnp.int32

_MESH = plsc.VectorSubcoreMesh(core_axis_name="c", subcore_axis_name="s")
_SC_PARAMS = pltpu.CompilerParams(
    needs_layout_passes=False, use_tc_tiling_on_sc=False
)


def _node_body(x_ref, wt_ref, a12_ref, h4_ref, src_ref):
    t = pl.program_id(0)
    x = x_ref[...]
    h = jnp.dot(x, wt_ref[...], preferred_element_type=f32)
    rows = t * 1024 + lax.broadcasted_iota(i32, (1024, 1), 0)
    h = jnp.where(rows < N, h, 0.0)
    for q in range(4):
        h4_ref[q, :, :] = h[:, 64 * q:64 * q + 64]
    src_ref[...] = jnp.dot(h, a12_ref[...], preferred_element_type=f32)


def _k_node(feats, wt, a12):
    return pl.pallas_call(
        _node_body,
        grid=(10,),
        in_specs=[
            pl.BlockSpec((1024, 256), lambda t: (t, 0)),
            pl.BlockSpec((256, 256), lambda t: (0, 0)),
            pl.BlockSpec((256, 16), lambda t: (0, 0)),
        ],
        out_specs=[
            pl.BlockSpec((4, 1024, 64), lambda t: (0, t, 0)),
            pl.BlockSpec((1024, 16), lambda t: (t, 0)),
        ],
        out_shape=[
            jax.ShapeDtypeStruct((4, NP, 64), f32),
            jax.ShapeDtypeStruct((NP, 16), f32),
        ],
    )(feats, wt, a12)


def _edge_body(x_ref, wt_ref, b8_ref, ea_ref, t_ref):
    ea = jnp.dot(x_ref[...], wt_ref[...], preferred_element_type=f32)
    ea_ref[...] = ea
    t_ref[...] = jnp.dot(ea, b8_ref[...], preferred_element_type=f32)


def _k_edge(edge_attr, wt, b8):
    return pl.pallas_call(
        _edge_body,
        grid=(80,),
        in_specs=[
            pl.BlockSpec((2000, 16), lambda t: (t, 0)),
            pl.BlockSpec((16, 16), lambda t: (0, 0)),
            pl.BlockSpec((16, 8), lambda t: (0, 0)),
        ],
        out_specs=[
            pl.BlockSpec((2000, 16), lambda t: (t, 0)),
            pl.BlockSpec((2000, 8), lambda t: (t, 0)),
        ],
        out_shape=[
            jax.ShapeDtypeStruct((E, 16), f32),
            jax.ShapeDtypeStruct((E, 8), f32),
        ],
    )(edge_attr, wt, b8)


def _rden_body(dp_ref, r_ref, rexp_ref):
    r = 1.0 / (dp_ref[0, :, :] + dp_ref[1, :, :])
    rexp_ref[...] = jnp.dot(r, r_ref[...], preferred_element_type=f32)


def _k_rden(dp3, rmat):
    return pl.pallas_call(
        _rden_body,
        grid=(10,),
        in_specs=[
            pl.BlockSpec((2, 1024, 8), lambda t: (0, t, 0)),
            pl.BlockSpec((8, 256), lambda t: (0, 0)),
        ],
        out_specs=pl.BlockSpec((1024, 256), lambda t: (t, 0)),
        out_shape=jax.ShapeDtypeStruct((NP, 256), f32),
    )(dp3, rmat)


def _out_body(a0_ref, a1_ref, a2_ref, a3_ref, rexp_ref, wt_ref, b_ref, bo_ref,
              o_ref):
    x = jnp.concatenate(
        [a0_ref[...], a1_ref[...], a2_ref[...], a3_ref[...]], axis=1)
    x = x * rexp_ref[...] + b_ref[...]
    o_ref[...] = jnp.dot(x, wt_ref[...], preferred_element_type=f32) + bo_ref[...]


def _k_out(a4, rexp, wt, b, bo):
    return pl.pallas_call(
        _out_body,
        grid=(10,),
        in_specs=[
            pl.BlockSpec((1024, 64), lambda t: (t, 0)),
            pl.BlockSpec((1024, 64), lambda t: (t, 0)),
            pl.BlockSpec((1024, 64), lambda t: (t, 0)),
            pl.BlockSpec((1024, 64), lambda t: (t, 0)),
            pl.BlockSpec((1024, 256), lambda t: (t, 0)),
            pl.BlockSpec((256, 256), lambda t: (0, 0)),
            pl.BlockSpec((1, 256), lambda t: (0, 0)),
            pl.BlockSpec((1, 256), lambda t: (0, 0)),
        ],
        out_specs=pl.BlockSpec((1024, 256), lambda t: (t, 0)),
        out_shape=jax.ShapeDtypeStruct((N, 256), f32),
    )(a4[0], a4[1], a4[2], a4[3], rexp, wt, b, bo)


NB_A = NBLK // 32
NB_G = NBLK // 16


def _attn_body(row_hbm, col_hbm, t_hbm, src_hbm, z8_hbm, ex_hbm, dp_hbm,
               rowslab, colflat, tslab, srgs, scgs, exflats, ex2ds, acc,
               gsems, ssems):
    c = lax.axis_index("c")
    s = lax.axis_index("s")
    w = c * 16 + s
    pltpu.sync_copy(z8_hbm.at[pl.ds(s * STRIPE, STRIPE), :],
                    acc.at[pl.ds(s * STRIPE, STRIPE), :])
    pltpu.sync_copy(row_hbm.at[pl.ds(w * NB_A, NB_A), :], rowslab)
    pltpu.sync_copy(col_hbm.at[pl.ds(w * NB_A, NB_A), :], colflat)
    pltpu.sync_copy(t_hbm.at[pl.ds(w * NB_A * 1024, NB_A * 1024)], tslab)
    plsc.subcore_barrier()

    iota = lax.iota(i32, 16)
    id8 = iota // 8
    im8 = iota % 8

    def g_descs(k, p):
        return [
            pltpu.make_async_copy(src_hbm.at[rowslab.at[k]], srgs[p], gsems[p]),
            pltpu.make_async_copy(src_hbm.at[colflat.at[k]], scgs[p], gsems[p]),
        ]

    def step(k, p):
        for d in g_descs(k, p):
            d.wait()

        @pl.loop(0, 64, unroll=4)
        def _vec(v):
            ir = 2 * v + id8
            srv = plsc.load_gather(srgs[p], [ir, im8])
            scv = plsc.load_gather(scgs[p], [ir, im8 + 8])
            tv = tslab[pl.ds(1024 * k + 16 * v, 16)]
            lg = srv + scv + tv
            exv = jnp.exp(jnp.maximum(lg, 0.01 * lg))
            exflats[0][pl.ds(16 * v, 16)] = exv
            plsc.store_scatter(ex2ds[0], [ir, im8], exv)

        pltpu.sync_copy(ex2ds[0], acc.at[rowslab.at[k]], add=True)
        pltpu.sync_copy(exflats[0],
                        ex_hbm.at[pl.ds((w * NB_A + k) * 1024, 1024)])
        knext = jnp.minimum(k + 2, NB_A - 1)
        for d in g_descs(knext, p):
            d.start()

    for p in range(2):
        for d in g_descs(p, p):
            d.start()

    @pl.loop(0, NB_A // 2)
    def _pair(kk):
        step(2 * kk, 0)
        step(2 * kk + 1, 1)

    for p in range(2):
        for d in g_descs(NB_A - 1, p):
            d.wait()

    plsc.subcore_barrier()
    pltpu.sync_copy(acc.at[pl.ds(s * STRIPE, STRIPE), :],
                    dp_hbm.at[c].at[pl.ds(s * STRIPE, STRIPE), :])


def _k_attn(row2d, col2d, tp, src_tab, z8):
    return pl.kernel(
        _attn_body,
        out_type=(
            jax.ShapeDtypeStruct((EP * 8,), f32),
            jax.ShapeDtypeStruct((2, NP, 8), f32),
        ),
        mesh=_MESH,
        compiler_params=_SC_PARAMS,
        scratch_types=[
            pltpu.VMEM((NB_A, 128), i32),
            pltpu.VMEM((NB_A, 128), i32),
            pltpu.VMEM((NB_A * 1024,), f32),
            [pltpu.VMEM((128, 16), f32)] * 2,
            [pltpu.VMEM((128, 16), f32)] * 2,
            [pltpu.VMEM((1024,), f32)] * 2,
            [pltpu.VMEM((128, 8), f32)] * 2,
            pltpu.VMEM_SHARED((NP, 8), f32),
            [pltpu.SemaphoreType.DMA] * 2,
            [pltpu.SemaphoreType.DMA] * 2,
        ],
    )(row2d, col2d, tp, src_tab, z8)


def _agg_body(row_hbm, col_hbm, ex_hbm, h4_hbm, z64_hbm, agg_hbm,
              rowslab, colflat2, hbufs, exbs, acc, gsems, ssems):
    c = lax.axis_index("c")
    s = lax.axis_index("s")
    pltpu.sync_copy(row_hbm.at[pl.ds(s * NB_G, NB_G), :], rowslab)
    pltpu.sync_copy(col_hbm.at[pl.ds(s * NB_G * 128, NB_G * 128)], colflat2)

    iota = lax.iota(i32, 16)
    izero = iota * 0

    off0 = 2 * c * NP

    @pl.loop(0, NB_G * 8, unroll=4)
    def _shift(v):
        colflat2[pl.ds(16 * v, 16)] = colflat2[pl.ds(16 * v, 16)] + off0

    for q in range(2):
        if q == 1:
            @pl.loop(0, NB_G * 8, unroll=4)
            def _shift2(v):
                colflat2[pl.ds(16 * v, 16)] = colflat2[pl.ds(16 * v, 16)] + NP

        pltpu.sync_copy(z64_hbm.at[pl.ds(s * STRIPE, STRIPE), :],
                        acc.at[pl.ds(s * STRIPE, STRIPE), :])
        plsc.subcore_barrier()

        def g_descs(k, p):
            return [
                pltpu.make_async_copy(
                    h4_hbm.at[colflat2.at[pl.ds(256 * k, 128)]],
                    hbufs[p].at[pl.ds(0, 128), :], gsems[p]),
                pltpu.make_async_copy(
                    h4_hbm.at[colflat2.at[pl.ds(256 * k + 128, 128)]],
                    hbufs[p].at[pl.ds(128, 128), :], gsems[p]),
                pltpu.make_async_copy(
                    ex_hbm.at[pl.ds((s * NB_G + 2 * k) * 1024, 2048)],
                    exbs[p], gsems[p]),
            ]

        h0 = 4 * c + 2 * q
        NS = NB_G // 2

        def step(k, p):
            for d in g_descs(k, p):
                d.wait()

            @pl.loop(0, 256, unroll=8)
            def _edge(i):
                base = 8 * i + h0
                for jj in range(2):
                    av = plsc.load_gather(exbs[p], [(base + jj) + izero])
                    lo = 32 * jj
                    hbufs[p][i, pl.ds(lo, 16)] = (
                        hbufs[p][i, pl.ds(lo, 16)] * av)
                    hbufs[p][i, pl.ds(lo + 16, 16)] = (
                        hbufs[p][i, pl.ds(lo + 16, 16)] * av)

            pltpu.sync_copy(hbufs[p].at[pl.ds(0, 128), :],
                            acc.at[rowslab.at[2 * k]], add=True)
            pltpu.sync_copy(hbufs[p].at[pl.ds(128, 128), :],
                            acc.at[rowslab.at[2 * k + 1]], add=True)
            knext = jnp.minimum(k + 2, NS - 1)
            for d in g_descs(knext, p):
                d.start()

        for p in range(2):
            for d in g_descs(p, p):
                d.start()

        @pl.loop(0, NS // 2)
        def _pair(kk):
            step(2 * kk, 0)
            step(2 * kk + 1, 1)

        for p in range(2):
            for d in g_descs(NS - 1, p):
                d.wait()

        plsc.subcore_barrier()
        pltpu.sync_copy(acc.at[pl.ds(s * STRIPE, STRIPE), :],
                        agg_hbm.at[2 * c + q].at[pl.ds(s * STRIPE, STRIPE), :])


def _k_agg(row2d, col_flat, ex_flat, h4flat, z64):
    return pl.kernel(
        _agg_body,
        out_type=jax.ShapeDtypeStruct((4, NP, 64), f32),
        mesh=_MESH,
        compiler_params=_SC_PARAMS,
        scratch_types=[
            pltpu.VMEM((NB_G, 128), i32),
            pltpu.VMEM((NB_G * 128,), i32),
            [pltpu.VMEM((256, 64), f32)] * 2,
            [pltpu.VMEM((2048,), f32)] * 2,
            pltpu.VMEM_SHARED((NP, 64), f32),
            [pltpu.SemaphoreType.DMA] * 2,
            [pltpu.SemaphoreType.DMA] * 2,
        ],
    )(row2d, col_flat, ex_flat, h4flat, z64)


@jax.jit
def kernel(feats, edge_index, edge_attr, W_fc, W_edge, att, bias, W_out, b_out):
    row = edge_index[:, 0]
    col = edge_index[:, 1]
    rowp = jnp.concatenate([row, jnp.full((EP - E,), N, i32)]).reshape(NBLK, 128)
    colp = jnp.concatenate([col, jnp.full((EP - E,), N, i32)]).reshape(NBLK, 128)

    att_f = att[..., 0]
    r256 = jnp.arange(256)
    a12 = (
        jnp.zeros((256, 16), f32)
        .at[r256, r256 // 32].set(att_f[:, :32].reshape(-1))
        .at[r256, 8 + r256 // 32].set(att_f[:, 32:64].reshape(-1))
    )
    r16 = jnp.arange(16)
    b8 = jnp.zeros((16, 8), f32).at[r16, r16 // 2].set(att_f[:, 64:66].reshape(-1))

    r64 = jnp.arange(256)
    rmat = jnp.zeros((8, 256), f32).at[r64 // 32, r64].set(1.0)

    h4, src_tab = _k_node(feats, W_fc.T, a12)
    ea16, t8 = _k_edge(edge_attr, W_edge.T, b8)
    tp = jnp.pad(t8.reshape(-1), (0, (EP - E) * 8))

    z8 = jnp.zeros((NP, 8), f32)
    z64 = jnp.zeros((NP, 64), f32)

    ex_flat, dpart = _k_attn(rowp, colp, tp, src_tab, z8)
    rexp = _k_rden(dpart, rmat)
    agg4 = _k_agg(rowp, colp.reshape(-1), ex_flat,
                  h4.reshape(4 * NP, 64), z64)

    out = _k_out(agg4, rexp, W_out.T, bias.reshape(1, 256),
                 b_out.reshape(1, 256))
    return out, edge_index, ea16
